# Initial kernel scaffold; baseline (speedup 1.0000x reference)
import jax
import jax.numpy as jnp
from jax.experimental import pallas as pl


def kernel(x, weight, p, a, b):
    raise NotImplementedError("write your pallas kernel here")


# speedup vs baseline: 12.4880x; 12.4880x over previous
import functools

import jax
import jax.numpy as jnp
from jax import lax
from jax.experimental import pallas as pl
from jax.experimental.pallas import tpu as pltpu
from jax.experimental.pallas import tpu_sc as plsc

_DIM = 32
_VCOMP = 100000
_NWORKERS = 32
_NTOK = 4096 * 26
_TOK_PER_W = _NTOK // _NWORKERS
_CHUNK = 1664
_NCHUNK = _TOK_PER_W // _CHUNK
_SEG = 128
_NSEG = _CHUNK // _SEG


def _hash_code(t_u, t_f, a_u, a_f, pinv_f, b_u, p_u):
    q = (t_f * a_f * pinv_f + jnp.float32(0.25)).astype(jnp.uint32)
    r = t_u * a_u + b_u - q * p_u
    r = jnp.where(r >= p_u, r + p_u, r)
    vc = jnp.uint32(_VCOMP)
    q2 = (r.astype(jnp.float32) * jnp.float32(1.0 / _VCOMP)
          + jnp.float32(0.25)).astype(jnp.uint32)
    code = r - q2 * vc
    code = jnp.where(code >= vc, code + vc, code)
    return code.astype(jnp.int32)


def _sc_body(x_ref, w_ref, cu_ref, cf_ref, out_ref,
             tok, idx0, idx1, rows0, rows1, cuv, cfv, sem):
    wid = lax.axis_index("s") * 2 + lax.axis_index("c")
    pltpu.sync_copy(cu_ref, cuv)
    pltpu.sync_copy(cf_ref, cfv)
    b0 = cuv[pl.ds(0, 16)]
    b1 = cuv[pl.ds(16, 16)]
    p0 = cuv[pl.ds(32, 16)]
    p1 = cuv[pl.ds(48, 16)]
    a0u = cuv[pl.ds(64, 16)]
    a1u = cuv[pl.ds(80, 16)]
    a0f = cfv[pl.ds(0, 16)]
    a1f = cfv[pl.ds(16, 16)]
    pi0 = cfv[pl.ds(32, 16)]
    pi1 = cfv[pl.ds(48, 16)]

    base_w = wid * jnp.int32(_TOK_PER_W)
    for c in range(_NCHUNK):
        tbase = base_w + jnp.int32(c * _CHUNK)
        pltpu.sync_copy(x_ref.at[pl.ds(tbase, _CHUNK)], tok)

        def hash_body(j, carry):
            for l in range(8):
                off = j * jnp.int32(_SEG) + jnp.int32(l * 16)
                t_u = tok[pl.ds(off, 16)]
                t_f = t_u.astype(jnp.float32)
                idx0[pl.ds(off, 16)] = _hash_code(t_u, t_f, a0u, a0f, pi0, b0, p0)
                idx1[pl.ds(off, 16)] = _hash_code(t_u, t_f, a1u, a1f, pi1, b1, p1)
            return carry

        lax.fori_loop(jnp.int32(0), jnp.int32(_NSEG), hash_body, jnp.int32(0))

        descs = []
        for j in range(_NSEG):
            s = pl.ds(j * _SEG, _SEG)
            descs.append(pltpu.async_copy(w_ref.at[idx0.at[s]], rows0.at[s], sem))
            descs.append(pltpu.async_copy(w_ref.at[idx1.at[s]], rows1.at[s], sem))
        for d in descs:
            d.wait()

        def mean_body(r, carry):
            for l in range(2):
                cs = pl.ds(l * 16, 16)
                v = (rows0[r, cs] + rows1[r, cs]) * jnp.float32(0.5)
                rows0[r, cs] = v
            return carry

        lax.fori_loop(jnp.int32(0), jnp.int32(_CHUNK), mean_body, jnp.int32(0))
        pltpu.sync_copy(rows0, out_ref.at[pl.ds(tbase, _CHUNK)])


@jax.jit
def _run(x_u32, weight, cu, cf):
    mesh = plsc.VectorSubcoreMesh(core_axis_name="c", subcore_axis_name="s")
    f = functools.partial(
        pl.kernel,
        out_type=jax.ShapeDtypeStruct((_NTOK, _DIM), jnp.float32),
        mesh=mesh,
        compiler_params=pltpu.CompilerParams(use_tc_tiling_on_sc=False),
        scratch_types=[
            pltpu.VMEM((_CHUNK,), jnp.uint32),
            pltpu.VMEM((_CHUNK,), jnp.int32),
            pltpu.VMEM((_CHUNK,), jnp.int32),
            pltpu.VMEM((_CHUNK, _DIM), jnp.float32),
            pltpu.VMEM((_CHUNK, _DIM), jnp.float32),
            pltpu.VMEM((96,), jnp.uint32),
            pltpu.VMEM((64,), jnp.float32),
            pltpu.SemaphoreType.DMA,
        ],
    )(_sc_body)
    return f(x_u32, weight, cu, cf)


def kernel(x, weight, p, a, b):
    x_shape = x.shape
    x_u32 = x.reshape(-1).astype(jnp.uint32)
    a_u = a.astype(jnp.uint32)
    b_u = b.astype(jnp.uint32)
    p_u = p.astype(jnp.uint32)
    a_f = a.astype(jnp.float32)
    pinv = jnp.float32(1.0) / p.astype(jnp.float32)
    cu = jnp.repeat(jnp.stack([b_u[0], b_u[1], p_u[0], p_u[1], a_u[0], a_u[1]]), 16)
    cf = jnp.repeat(jnp.stack([a_f[0], a_f[1], pinv[0], pinv[1]]), 16)
    out = _run(x_u32, weight, cu, cf)
    return out.reshape(x_shape + (_DIM,))


# speedup vs baseline: 14.9218x; 1.1949x over previous
import functools

import jax
import jax.numpy as jnp
from jax import lax
from jax.experimental import pallas as pl
from jax.experimental.pallas import tpu as pltpu
from jax.experimental.pallas import tpu_sc as plsc

_DIM = 32
_VCOMP = 100000
_I0 = 4096
_I1 = 26
_NTOK = _I0 * _I1
_NW = 32
_TPW = _NTOK // _NW
_G = 2
_NCH = _I1 // _G
_CT = _G * 128
_ORWS = _I1 * _DIM * (_I0 // 128)
_CROWS = _G * _DIM


def _iota16():
    return lax.broadcasted_iota(jnp.int32, (16,), 0)


def _hash_code(t_u, t_f, a_u, a_f, pinv_f, b_u, p_u):
    q = (t_f * a_f * pinv_f + jnp.float32(0.25)).astype(jnp.uint32)
    r = t_u * a_u + b_u - q * p_u
    r = jnp.where(r >= p_u, r + p_u, r)
    vc = jnp.uint32(_VCOMP)
    q2 = (r.astype(jnp.float32) * jnp.float32(1.0 / _VCOMP)
          + jnp.float32(0.25)).astype(jnp.uint32)
    code = r - q2 * vc
    code = jnp.where(code >= vc, code + vc, code)
    return code.astype(jnp.int32)


def _sc_body(x_ref, w_ref, cu_ref, cf_ref, out_ref,
             tok, idx0a, idx1a, idx0b, idx1b, rows0a, rows1a, rows0b, rows1b,
             ostga, ostgb, sidx, cuv, cfv, sem, sem_out):
    wid = lax.axis_index("s") * 2 + lax.axis_index("c")
    pltpu.sync_copy(cu_ref, cuv)
    pltpu.sync_copy(cf_ref, cfv)
    b0 = cuv[pl.ds(0, 16)]
    b1 = cuv[pl.ds(16, 16)]
    p0 = cuv[pl.ds(32, 16)]
    p1 = cuv[pl.ds(48, 16)]
    a0u = cuv[pl.ds(64, 16)]
    a1u = cuv[pl.ds(80, 16)]
    a0f = cfv[pl.ds(0, 16)]
    a1f = cfv[pl.ds(16, 16)]
    pi0 = cfv[pl.ds(32, 16)]
    pi1 = cfv[pl.ds(48, 16)]

    iota = _iota16()
    iota26 = iota * jnp.int32(_I1)
    iota32 = iota * jnp.int32(_DIM)

    idx = [(idx0a, idx1a), (idx0b, idx1b)]
    rows = [(rows0a, rows1a), (rows0b, rows1b)]
    ostg = [ostga, ostgb]

    pltpu.sync_copy(x_ref.at[pl.ds(wid * jnp.int32(_TPW), _TPW)], tok)

    for c in range(_NCH):
        for v in range(4):
            base = jnp.int32(2048 * c + 512 * v) + wid
            sidx[jnp.int32(c), pl.ds(v * 16, 16)] = base + iota32

    def hash_chunk(c, par):
        i0x, i1x = idx[par]

        def hash_body(j, carry):
            i1 = jnp.int32(2 * c) + lax.shift_right_logical(j, jnp.int32(3))
            k = lax.bitwise_and(j, jnp.int32(7))
            tokidx = iota26 + k * jnp.int32(16 * _I1) + i1
            t_i = plsc.load_gather(tok, [tokidx])
            t_u = t_i.astype(jnp.uint32)
            t_f = t_i.astype(jnp.float32)
            off = j * jnp.int32(16)
            i0x[pl.ds(off, 16)] = _hash_code(t_u, t_f, a0u, a0f, pi0, b0, p0)
            i1x[pl.ds(off, 16)] = _hash_code(t_u, t_f, a1u, a1f, pi1, b1, p1)
            return carry

        lax.fori_loop(jnp.int32(0), jnp.int32(_CT // 16), hash_body,
                      jnp.int32(0))

    def fire_gathers(par):
        i0x, i1x = idx[par]
        r0x, r1x = rows[par]
        descs = []
        for s in range(_CT // 128):
            sl = pl.ds(s * 128, 128)
            descs.append(pltpu.async_copy(w_ref.at[i0x.at[sl]], r0x.at[sl], sem))
            descs.append(pltpu.async_copy(w_ref.at[i1x.at[sl]], r1x.at[sl], sem))
        return descs

    def mean_chunk(par):
        r0x, r1x = rows[par]
        og = ostg[par]

        def mean_body(m, carry):
            gl = lax.shift_right_logical(m, jnp.int32(4))
            k = lax.bitwise_and(lax.shift_right_logical(m, jnp.int32(1)),
                                jnp.int32(7))
            dh = lax.bitwise_and(m, jnp.int32(1))
            rbase = gl * jnp.int32(128) + k * jnp.int32(16) + iota
            dbase = dh * jnp.int32(16)
            orow0 = gl * jnp.int32(_DIM) + dbase
            ocol = k * jnp.int32(16) + iota
            for r in range(16):
                rot = lax.bitwise_and(iota + jnp.int32(r), jnp.int32(15))
                dcol = dbase + rot
                v0 = plsc.load_gather(r0x, [rbase, dcol])
                v1 = plsc.load_gather(r1x, [rbase, dcol])
                vm = (v0 + v1) * jnp.float32(0.5)
                plsc.store_scatter(og, [orow0 + rot, ocol], vm)
            return carry

        lax.fori_loop(jnp.int32(0), jnp.int32(32), mean_body, jnp.int32(0))

    gdescs = [None, None]
    out_descs = [None, None]
    for c in range(_NCH + 1):
        par = c & 1
        if c < _NCH:
            hash_chunk(c, par)
            gdescs[par] = fire_gathers(par)
        if c >= 1:
            prev = 1 - par
            for d in gdescs[prev]:
                d.wait()
            if out_descs[prev] is not None:
                out_descs[prev].wait()
            mean_chunk(prev)
            out_descs[prev] = pltpu.async_copy(
                ostg[prev], out_ref.at[sidx.at[jnp.int32(c - 1)]], sem_out)

    for d in out_descs:
        if d is not None:
            d.wait()


@jax.jit
def _run(x_i32, weight, cu, cf):
    mesh = plsc.VectorSubcoreMesh(core_axis_name="c", subcore_axis_name="s")
    f = functools.partial(
        pl.kernel,
        out_type=jax.ShapeDtypeStruct((_ORWS, 128), jnp.float32),
        mesh=mesh,
        compiler_params=pltpu.CompilerParams(use_tc_tiling_on_sc=False,
                                             needs_layout_passes=False),
        scratch_types=[
            pltpu.VMEM((_TPW,), jnp.int32),
            pltpu.VMEM((_CT,), jnp.int32),
            pltpu.VMEM((_CT,), jnp.int32),
            pltpu.VMEM((_CT,), jnp.int32),
            pltpu.VMEM((_CT,), jnp.int32),
            pltpu.VMEM((_CT, _DIM), jnp.float32),
            pltpu.VMEM((_CT, _DIM), jnp.float32),
            pltpu.VMEM((_CT, _DIM), jnp.float32),
            pltpu.VMEM((_CT, _DIM), jnp.float32),
            pltpu.VMEM((_CROWS, 128), jnp.float32),
            pltpu.VMEM((_CROWS, 128), jnp.float32),
            pltpu.VMEM((_NCH, _CROWS), jnp.int32),
            pltpu.VMEM((96,), jnp.uint32),
            pltpu.VMEM((64,), jnp.float32),
            pltpu.SemaphoreType.DMA,
            pltpu.SemaphoreType.DMA,
        ],
    )(_sc_body)
    return f(x_i32, weight, cu, cf)


def kernel(x, weight, p, a, b):
    x_shape = x.shape
    x_i32 = x.reshape(-1).astype(jnp.int32)
    a_u = a.astype(jnp.uint32)
    b_u = b.astype(jnp.uint32)
    p_u = p.astype(jnp.uint32)
    a_f = a.astype(jnp.float32)
    pinv = jnp.float32(1.0) / p.astype(jnp.float32)
    cu = jnp.repeat(jnp.stack([b_u[0], b_u[1], p_u[0], p_u[1], a_u[0], a_u[1]]), 16)
    cf = jnp.repeat(jnp.stack([a_f[0], a_f[1], pinv[0], pinv[1]]), 16)
    out2 = _run(x_i32, weight, cu, cf)
    out3 = out2.reshape(_I1, _DIM, _I0)
    return jnp.transpose(out3, (2, 0, 1)).reshape(x_shape + (_DIM,))


# speedup vs baseline: 19.2090x; 1.2873x over previous
import functools

import jax
import jax.numpy as jnp
from jax import lax
from jax.experimental import pallas as pl
from jax.experimental.pallas import tpu as pltpu
from jax.experimental.pallas import tpu_sc as plsc

_DIM = 32
_VCOMP = 100000
_I0 = 4096
_I1 = 26
_NTOK = _I0 * _I1
_NW = 32
_TPW = _NTOK // _NW
_GMAX = 4
_CHUNKS = [(0, 4), (4, 4), (8, 4), (12, 4), (16, 4), (20, 4), (24, 2)]
_CT = _GMAX * 128
_ORWS = _I1 * _DIM * (_I0 // 128)
_CROWS = _GMAX * _DIM


def _iota16():
    return lax.broadcasted_iota(jnp.int32, (16,), 0)


def _hash_code(t_u, t_f, a_u, a_f, pinv_f, b_u, p_u):
    q = (t_f * a_f * pinv_f + jnp.float32(0.25)).astype(jnp.uint32)
    r = t_u * a_u + b_u - q * p_u
    r = jnp.where(r >= p_u, r + p_u, r)
    vc = jnp.uint32(_VCOMP)
    q2 = (r.astype(jnp.float32) * jnp.float32(1.0 / _VCOMP)
          + jnp.float32(0.25)).astype(jnp.uint32)
    code = r - q2 * vc
    code = jnp.where(code >= vc, code + vc, code)
    return code.astype(jnp.int32)


def _sc_body(x_ref, w_ref, cu_ref, cf_ref, out_ref,
             tok, idx0a, idx1a, idx0b, idx1b, rows0a, rows1a, rows0b, rows1b,
             ostga, ostgb, sidx, cuv, cfv, sem, sem_out):
    wid = lax.axis_index("s") * 2 + lax.axis_index("c")
    pltpu.sync_copy(cu_ref, cuv)
    pltpu.sync_copy(cf_ref, cfv)
    b0 = cuv[pl.ds(0, 16)]
    b1 = cuv[pl.ds(16, 16)]
    p0 = cuv[pl.ds(32, 16)]
    p1 = cuv[pl.ds(48, 16)]
    a0u = cuv[pl.ds(64, 16)]
    a1u = cuv[pl.ds(80, 16)]
    a0f = cfv[pl.ds(0, 16)]
    a1f = cfv[pl.ds(16, 16)]
    pi0 = cfv[pl.ds(32, 16)]
    pi1 = cfv[pl.ds(48, 16)]

    iota = _iota16()
    iota26 = iota * jnp.int32(_I1)
    iota32 = iota * jnp.int32(_DIM)

    idx = [(idx0a, idx1a), (idx0b, idx1b)]
    rows = [(rows0a, rows1a), (rows0b, rows1b)]
    ostg = [ostga, ostgb]

    pltpu.sync_copy(x_ref.at[pl.ds(wid * jnp.int32(_TPW), _TPW)], tok)

    for c in range(_I1 // 2):
        for v in range(4):
            base = jnp.int32(2048 * c + 512 * v) + wid
            sidx[jnp.int32(c), pl.ds(v * 16, 16)] = base + iota32

    def hash_chunk(g0, ng, par):
        i0x, i1x = idx[par]

        @plsc.parallel_loop(jnp.int32(0), jnp.int32(ng * 8), jnp.int32(1), unroll=1)
        def hash_body(j):
            i1 = jnp.int32(g0) + lax.shift_right_logical(j, jnp.int32(3))
            k = lax.bitwise_and(j, jnp.int32(7))
            tokidx = iota26 + k * jnp.int32(16 * _I1) + i1
            t_i = plsc.load_gather(tok, [tokidx])
            t_u = t_i.astype(jnp.uint32)
            t_f = t_i.astype(jnp.float32)
            off = j * jnp.int32(16)
            i0x[pl.ds(off, 16)] = _hash_code(t_u, t_f, a0u, a0f, pi0, b0, p0)
            i1x[pl.ds(off, 16)] = _hash_code(t_u, t_f, a1u, a1f, pi1, b1, p1)

    def fire_gathers(ng, par):
        i0x, i1x = idx[par]
        r0x, r1x = rows[par]
        descs = []
        for s in range(ng):
            sl = pl.ds(s * 128, 128)
            descs.append(pltpu.async_copy(w_ref.at[i0x.at[sl]], r0x.at[sl], sem))
            descs.append(pltpu.async_copy(w_ref.at[i1x.at[sl]], r1x.at[sl], sem))
        return descs

    def mean_chunk(ng, par):
        r0x, r1x = rows[par]
        og = ostg[par]

        @plsc.parallel_loop(jnp.int32(0), jnp.int32(ng * 16), jnp.int32(1), unroll=1)
        def mean_body(m):
            gl = lax.shift_right_logical(m, jnp.int32(4))
            k = lax.bitwise_and(lax.shift_right_logical(m, jnp.int32(1)),
                                jnp.int32(7))
            dh = lax.bitwise_and(m, jnp.int32(1))
            rbase = gl * jnp.int32(128) + k * jnp.int32(16) + iota
            dbase = dh * jnp.int32(16)
            orow0 = gl * jnp.int32(_DIM) + dbase
            ocol = k * jnp.int32(16) + iota
            for r in range(16):
                rot = lax.bitwise_and(iota + jnp.int32(r), jnp.int32(15))
                dcol = dbase + rot
                v0 = plsc.load_gather(r0x, [rbase, dcol])
                v1 = plsc.load_gather(r1x, [rbase, dcol])
                vm = (v0 + v1) * jnp.float32(0.5)
                plsc.store_scatter(og, [orow0 + rot, ocol], vm)

    nch = len(_CHUNKS)
    gdescs = [None, None]
    out_descs = [None, None]
    for c in range(nch + 1):
        par = c & 1
        if c < nch:
            g0, ng = _CHUNKS[c]
            hash_chunk(g0, ng, par)
            gdescs[par] = fire_gathers(ng, par)
        if c >= 1:
            prev = 1 - par
            pg0, png = _CHUNKS[c - 1]
            for d in gdescs[prev]:
                d.wait()
            if out_descs[prev] is not None:
                for d in out_descs[prev]:
                    d.wait()
            mean_chunk(png, prev)
            odescs = []
            for t in range(png // 2):
                odescs.append(pltpu.async_copy(
                    ostg[prev].at[pl.ds(t * 64, 64)],
                    out_ref.at[sidx.at[jnp.int32(pg0 // 2 + t)]], sem_out))
            out_descs[prev] = odescs

    for ds_ in out_descs:
        if ds_ is not None:
            for d in ds_:
                d.wait()


@jax.jit
def _run(x_i32, weight, cu, cf):
    mesh = plsc.VectorSubcoreMesh(core_axis_name="c", subcore_axis_name="s")
    f = functools.partial(
        pl.kernel,
        out_type=jax.ShapeDtypeStruct((_ORWS, 128), jnp.float32),
        mesh=mesh,
        compiler_params=pltpu.CompilerParams(use_tc_tiling_on_sc=False,
                                             needs_layout_passes=False),
        scratch_types=[
            pltpu.VMEM((_TPW,), jnp.int32),
            pltpu.VMEM((_CT,), jnp.int32),
            pltpu.VMEM((_CT,), jnp.int32),
            pltpu.VMEM((_CT,), jnp.int32),
            pltpu.VMEM((_CT,), jnp.int32),
            pltpu.VMEM((_CT, _DIM), jnp.float32),
            pltpu.VMEM((_CT, _DIM), jnp.float32),
            pltpu.VMEM((_CT, _DIM), jnp.float32),
            pltpu.VMEM((_CT, _DIM), jnp.float32),
            pltpu.VMEM((_CROWS, 128), jnp.float32),
            pltpu.VMEM((_CROWS, 128), jnp.float32),
            pltpu.VMEM((_I1 // 2, 64), jnp.int32),
            pltpu.VMEM((96,), jnp.uint32),
            pltpu.VMEM((64,), jnp.float32),
            pltpu.SemaphoreType.DMA,
            pltpu.SemaphoreType.DMA,
        ],
    )(_sc_body)
    return f(x_i32, weight, cu, cf)


def kernel(x, weight, p, a, b):
    x_shape = x.shape
    x_i32 = x.reshape(-1).astype(jnp.int32)
    a_u = a.astype(jnp.uint32)
    b_u = b.astype(jnp.uint32)
    p_u = p.astype(jnp.uint32)
    a_f = a.astype(jnp.float32)
    pinv = jnp.float32(1.0) / p.astype(jnp.float32)
    cu = jnp.repeat(jnp.stack([b_u[0], b_u[1], p_u[0], p_u[1], a_u[0], a_u[1]]), 16)
    cf = jnp.repeat(jnp.stack([a_f[0], a_f[1], pinv[0], pinv[1]]), 16)
    out2 = _run(x_i32, weight, cu, cf)
    out3 = out2.reshape(_I1, _DIM, _I0)
    return jnp.transpose(out3, (2, 0, 1)).reshape(x_shape + (_DIM,))
